# baseline (speedup 1.0000x reference)
import functools

import numpy as np
import jax
import jax.numpy as jnp
from jax import lax
from jax.experimental import pallas as pl
from jax.experimental.pallas import tpu as pltpu
from jax.experimental.pallas import tpu_sc as plsc

F = 26
V = 100000
K = 16
B = 16384
D = 13

NC = 2
NS = 16
NW = NC * NS

ROWS = B * F
PLANE = B * 128 // K

FA = 24
FB = F - FA
GA = 3
GB = 1
CH = 1024
NJ = CH // 128

LCH = 1664
LNJ = LCH // 128
LNCH = 8
SPW = B // NW
SPC = LCH // F
LTAB = 784 * 128


def _xpose_body(qf_ref, out_ref):
    x = qf_ref[...]
    for t in range(x.shape[1] // 128):
        out_ref[0, pl.ds(128 * t, 128), :] = x[:, 128 * t:128 * (t + 1)].T


def _lin_body(l_ref, out_ref):
    x = l_ref[0, 0, :]
    out_ref[0, :781, :] = x[:781 * 128].reshape(781, 128)
    out_ref[0, 781, :32] = x[781 * 128:]


def _half_scatter_rows(f_lo: int, f_hi: int) -> np.ndarray:
    n = (f_hi - f_lo) * B
    r = np.arange(n, dtype=np.int64)
    f = r // B + f_lo
    b = r % B
    p = ((f - f_lo) // 8) * PLANE + b * 8 + (f % 8)
    return p.astype(np.int32).reshape(n // 128, 128)


_SIDX_A = _half_scatter_rows(0, FA)
_SIDX_B = _half_scatter_rows(FA, F)


def _fm_chunks(fidx_v, sidx_v, fm_tab, fm_v, fm_out, sem_f, sem_s, nch):
    def chunk(c, carry):
        gcs = [pltpu.async_copy(fm_tab.at[fidx_v.at[c * NJ + j]],
                                fm_v.at[pl.ds(j * 128, 128)], sem_f)
               for j in range(NJ)]
        scs = []
        for j in range(NJ):
            gcs[j].wait()
            scs.append(pltpu.async_copy(fm_v.at[pl.ds(j * 128, 128)],
                                        fm_out.at[sidx_v.at[c * NJ + j]],
                                        sem_s))
        for cp in scs:
            cp.wait()
        return carry

    lax.fori_loop(0, nch, chunk, 0)


@functools.cache
def _make_sc_gather_a():
    rpw = FA * B // NW
    irows = rpw // 128

    @functools.partial(
        pl.kernel,
        mesh=plsc.VectorSubcoreMesh(core_axis_name="c", subcore_axis_name="s"),
        out_type=jax.ShapeDtypeStruct((GA * PLANE, K), jnp.float32),
        scratch_types=[
            pltpu.VMEM((irows, 128), jnp.int32),
            pltpu.VMEM((irows, 128), jnp.int32),
            pltpu.VMEM((CH, K), jnp.float32),
            pltpu.SemaphoreType.DMA,
            pltpu.SemaphoreType.DMA,
        ],
        compiler_params=pltpu.CompilerParams(use_tc_tiling_on_sc=False),
    )
    def _gather_a(fidx_hbm, sidx_hbm, fm_tab, fm_out,
                  fidx_v, sidx_v, fm_v, sem_f, sem_s):
        wid = lax.axis_index("s") * NC + lax.axis_index("c")
        irow0 = wid * irows
        pltpu.sync_copy(fidx_hbm.at[pl.ds(irow0, irows)], fidx_v)
        pltpu.sync_copy(sidx_hbm.at[pl.ds(irow0, irows)], sidx_v)
        _fm_chunks(fidx_v, sidx_v, fm_tab, fm_v, fm_out, sem_f, sem_s,
                   rpw // CH)

    return _gather_a


@functools.cache
def _make_sc_gather_b():
    rpw = FB * B // NW
    irows = rpw // 128
    lirows = ROWS // NW // 128

    @functools.partial(
        pl.kernel,
        mesh=plsc.VectorSubcoreMesh(core_axis_name="c", subcore_axis_name="s"),
        out_type=[
            jax.ShapeDtypeStruct((GB * PLANE, K), jnp.float32),
            jax.ShapeDtypeStruct((B,), jnp.float32),
        ],
        scratch_types=[
            pltpu.VMEM((irows, 128), jnp.int32),
            pltpu.VMEM((irows, 128), jnp.int32),
            pltpu.VMEM((lirows, 128), jnp.int32),
            pltpu.VMEM((CH, K), jnp.float32),
            pltpu.VMEM((LCH,), jnp.float32),
            pltpu.VMEM((SPW,), jnp.float32),
            pltpu.SemaphoreType.DMA,
            pltpu.SemaphoreType.DMA,
            pltpu.SemaphoreType.DMA,
        ],
        compiler_params=pltpu.CompilerParams(use_tc_tiling_on_sc=False),
    )
    def _gather_b(fidx_hbm, sidx_hbm, lidx_hbm, fm_tab, lin_tab,
                  fm_out, lin_out, fidx_v, sidx_v, lidx_v, fm_v, lin_v,
                  ls_v, sem_f, sem_l, sem_s):
        wid = lax.axis_index("s") * NC + lax.axis_index("c")
        pltpu.sync_copy(fidx_hbm.at[pl.ds(wid * irows, irows)], fidx_v)
        pltpu.sync_copy(sidx_hbm.at[pl.ds(wid * irows, irows)], sidx_v)
        pltpu.sync_copy(lidx_hbm.at[pl.ds(wid * lirows, lirows)], lidx_v)
        _fm_chunks(fidx_v, sidx_v, fm_tab, fm_v, fm_out, sem_f, sem_s,
                   rpw // CH)

        def lin_chunk(c, carry):
            gcs = [pltpu.async_copy(lin_tab.at[lidx_v.at[c * LNJ + j]],
                                    lin_v.at[pl.ds(j * 128, 128)], sem_l)
                   for j in range(LNJ)]
            for cp in gcs:
                cp.wait()
            for g in range(SPC // 16):
                acc = jnp.zeros((16,), jnp.float32)
                for f in range(F):
                    acc = acc + lin_v[pl.ds(f * SPC + g * 16, 16)]
                ls_v[pl.ds(c * SPC + g * 16, 16)] = acc
            return carry

        lax.fori_loop(0, LNCH, lin_chunk, 0)
        pltpu.sync_copy(ls_v, lin_out.at[pl.ds(wid * SPW, SPW)])

    return _gather_b


def _tc_body(fa_ref, fb_ref, xdt_ref, lin_ref, w0a_ref, w0b_ref, b0_ref,
             w1_ref, b1_ref, wo_ref, wd_ref, bd_ref, s_ref, out_ref):
    ya = fa_ref[...]
    yb = fb_ref[...]
    xdt = xdt_ref[...]
    w0a = w0a_ref[...]
    s_mat = s_ref[...]
    lane = lax.broadcasted_iota(jnp.int32, ya.shape[1:], 1)
    dn = (((0,), (0,)), ((), ()))
    h0 = lax.dot_general(xdt, w0b_ref[...], dimension_numbers=dn,
                         preferred_element_type=jnp.float32)
    sums = jnp.zeros((ya.shape[1], K), jnp.float32)
    sos = jnp.zeros((ya.shape[1], K), jnp.float32)
    planes = [ya[0], ya[1], ya[2], jnp.where(lane < 32, yb[0], 0.0)]
    for ct in range(4):
        x_ct = planes[ct]
        w_ct = w0a[128 * ct:128 * (ct + 1), :]
        s_ct = s_mat[128 * ct:128 * (ct + 1), :]
        h0 = h0 + jnp.dot(x_ct, w_ct, preferred_element_type=jnp.float32)
        sums = sums + jnp.dot(x_ct, s_ct, preferred_element_type=jnp.float32)
        sos = sos + jnp.dot(x_ct * x_ct, s_ct,
                            preferred_element_type=jnp.float32)
    h0 = jnp.maximum(h0 + b0_ref[...], 0.0)
    h1 = jnp.dot(h0, w1_ref[...], preferred_element_type=jnp.float32)
    h1 = jnp.maximum(h1 + b1_ref[...], 0.0)
    dnn = jnp.sum(h1 * wo_ref[...], axis=1, keepdims=True)
    cross = 0.5 * jnp.sum(sums * sums - sos, axis=1, keepdims=True)
    lind = lax.dot_general(xdt, wd_ref[...], dimension_numbers=dn,
                           preferred_element_type=jnp.float32)
    logit = lin_ref[...] + lind + cross + dnn + bd_ref[0, 0]
    out_ref[...] = jax.nn.sigmoid(logit)


def kernel(X_sparse, X_dense, lin_emb, fm_emb, W_dense, b_dense,
           W0, b0, W1, b1, W_out):
    xs = X_sparse.astype(jnp.int32)
    f_arange = np.arange(F, dtype=np.int32)
    g_local = np.where(f_arange < FA, f_arange // 8, (f_arange - FA) // 8)
    qoff = jnp.asarray(g_local.astype(np.int32) * (8 * V) + (f_arange % 8))
    flat_idx = (xs.T * 8 + qoff[:, None]).reshape(ROWS // 128, 128)
    fidx_a = flat_idx[:FA * B // 128]
    fidx_b = flat_idx[FA * B // 128:]
    loffs = jnp.arange(F, dtype=jnp.int32) * LTAB
    lin_idx = (xs.reshape(NW, LNCH, SPC, F).transpose(0, 1, 3, 2)
               + loffs[None, None, :, None]).reshape(ROWS // 128, 128)

    qf = fm_emb.transpose(0, 2, 1).reshape(F * K, V)
    tab_a = pl.pallas_call(
        _xpose_body,
        grid=(GA, 13),
        in_specs=[pl.BlockSpec((128, 8192), lambda q, c: (q, c))],
        out_specs=pl.BlockSpec((1, 8192, 128), lambda q, c: (q, c, 0)),
        out_shape=jax.ShapeDtypeStruct((GA, V, 128), jnp.float32),
    )(qf).reshape(GA * V * 8, K)
    tab_b = pl.pallas_call(
        _xpose_body,
        grid=(GB, 13),
        in_specs=[pl.BlockSpec((128, 8192), lambda q, c: (q + GA, c))],
        out_specs=pl.BlockSpec((1, 8192, 128), lambda q, c: (q, c, 0)),
        out_shape=jax.ShapeDtypeStruct((GB, V, 128), jnp.float32),
    )(qf).reshape(GB * V * 8, K)
    l3 = lin_emb.transpose(0, 2, 1)
    lin_tab = pl.pallas_call(
        _lin_body,
        grid=(F,),
        in_specs=[pl.BlockSpec((1, 1, V), lambda f: (f, 0, 0))],
        out_specs=pl.BlockSpec((1, 784, 128), lambda f: (f, 0, 0)),
        out_shape=jax.ShapeDtypeStruct((F, 784, 128), jnp.float32),
    )(l3).reshape(F * LTAB)

    planes_a = _make_sc_gather_a()(fidx_a, jnp.asarray(_SIDX_A), tab_a)
    planes_b, lin_sum = _make_sc_gather_b()(
        fidx_b, jnp.asarray(_SIDX_B), lin_idx, tab_b, lin_tab)

    w0a = jnp.pad(W0[:, :F * K].T, ((0, 96), (0, 0)))
    w0b = W0[:, F * K:].T
    w1 = W1.T
    s_mat = jnp.asarray(np.vstack([
        np.tile(np.eye(K, dtype=np.float32), (F, 1)),
        np.zeros((96, K), np.float32)]))

    bB = 2048
    nb = B // bB
    out = pl.pallas_call(
        _tc_body,
        grid=(nb,),
        in_specs=[
            pl.BlockSpec((GA, bB, 128), lambda i: (0, i, 0)),
            pl.BlockSpec((GB, bB, 128), lambda i: (0, i, 0)),
            pl.BlockSpec((D, bB), lambda i: (0, i)),
            pl.BlockSpec((bB, 1), lambda i: (i, 0)),
            pl.BlockSpec((512, 256), lambda i: (0, 0)),
            pl.BlockSpec((D, 256), lambda i: (0, 0)),
            pl.BlockSpec((1, 256), lambda i: (0, 0)),
            pl.BlockSpec((256, 128), lambda i: (0, 0)),
            pl.BlockSpec((1, 128), lambda i: (0, 0)),
            pl.BlockSpec((1, 128), lambda i: (0, 0)),
            pl.BlockSpec((D, 1), lambda i: (0, 0)),
            pl.BlockSpec((1, 1), lambda i: (0, 0)),
            pl.BlockSpec((512, K), lambda i: (0, 0)),
        ],
        out_specs=pl.BlockSpec((bB, 1), lambda i: (i, 0)),
        out_shape=jax.ShapeDtypeStruct((B, 1), jnp.float32),
    )(planes_a.reshape(GA, B, 128), planes_b.reshape(GB, B, 128), X_dense.T,
      lin_sum.reshape(B, 1), w0a, w0b, b0.reshape(1, 256), w1,
      b1.reshape(1, 128), W_out, W_dense.T, b_dense.reshape(1, 1), s_mat)
    return out.reshape(B)


# speedup vs baseline: 1.0466x; 1.0466x over previous
import functools

import numpy as np
import jax
import jax.numpy as jnp
from jax import lax
from jax.experimental import pallas as pl
from jax.experimental.pallas import tpu as pltpu
from jax.experimental.pallas import tpu_sc as plsc

F = 26
V = 100000
K = 16
B = 16384
D = 13

NC = 2
NS = 16
NW = NC * NS

ROWS = B * F
PLANE = B * 128 // K
HROWS = 2 * PLANE

FA = 16
FB = F - FA
CH = 1024
NJ = CH // 128

LCH = 1664
LNJ = LCH // 128
LNCH = 8
SPW = B // NW
SPC = LCH // F
LTAB = 784 * 128


def _xpose_body(qf_ref, out_ref):
    x = qf_ref[...]
    for t in range(x.shape[1] // 128):
        out_ref[0, pl.ds(128 * t, 128), :] = x[:, 128 * t:128 * (t + 1)].T


def _lin_body(l_ref, out_ref):
    x = l_ref[0, 0, :]
    out_ref[0, :781, :] = x[:781 * 128].reshape(781, 128)
    out_ref[0, 781, :32] = x[781 * 128:]


def _half_scatter_rows(f_lo: int, f_hi: int) -> np.ndarray:
    n = (f_hi - f_lo) * B
    r = np.arange(n, dtype=np.int64)
    f = r // B + f_lo
    b = r % B
    p = ((f // 8) % 2) * PLANE + b * 8 + (f % 8)
    return p.astype(np.int32).reshape(n // 128, 128)


_SIDX_A = _half_scatter_rows(0, FA)
_SIDX_B = _half_scatter_rows(FA, F)


def _fm_chunks(fidx_v, sidx_v, fm_tab, fm_v, fm_out, sem_f, sem_s, nch):
    def chunk(c, carry):
        gcs = [pltpu.async_copy(fm_tab.at[fidx_v.at[c * NJ + j]],
                                fm_v.at[pl.ds(j * 128, 128)], sem_f)
               for j in range(NJ)]
        scs = []
        for j in range(NJ):
            gcs[j].wait()
            scs.append(pltpu.async_copy(fm_v.at[pl.ds(j * 128, 128)],
                                        fm_out.at[sidx_v.at[c * NJ + j]],
                                        sem_s))
        for cp in scs:
            cp.wait()
        return carry

    lax.fori_loop(0, nch, chunk, 0)


@functools.cache
def _make_sc_gather_a():
    rpw = FA * B // NW
    irows = rpw // 128
    lirows = ROWS // NW // 128

    @functools.partial(
        pl.kernel,
        mesh=plsc.VectorSubcoreMesh(core_axis_name="c", subcore_axis_name="s"),
        out_type=[
            jax.ShapeDtypeStruct((HROWS, K), jnp.float32),
            jax.ShapeDtypeStruct((B,), jnp.float32),
        ],
        scratch_types=[
            pltpu.VMEM((irows, 128), jnp.int32),
            pltpu.VMEM((irows, 128), jnp.int32),
            pltpu.VMEM((lirows, 128), jnp.int32),
            pltpu.VMEM((CH, K), jnp.float32),
            pltpu.VMEM((LCH,), jnp.float32),
            pltpu.VMEM((SPW,), jnp.float32),
            pltpu.SemaphoreType.DMA,
            pltpu.SemaphoreType.DMA,
            pltpu.SemaphoreType.DMA,
        ],
        compiler_params=pltpu.CompilerParams(use_tc_tiling_on_sc=False),
    )
    def _gather_a(fidx_hbm, sidx_hbm, lidx_hbm, fm_tab, lin_tab,
                  fm_out, lin_out, fidx_v, sidx_v, lidx_v, fm_v, lin_v,
                  ls_v, sem_f, sem_l, sem_s):
        wid = lax.axis_index("s") * NC + lax.axis_index("c")
        pltpu.sync_copy(fidx_hbm.at[pl.ds(wid * irows, irows)], fidx_v)
        pltpu.sync_copy(sidx_hbm.at[pl.ds(wid * irows, irows)], sidx_v)
        pltpu.sync_copy(lidx_hbm.at[pl.ds(wid * lirows, lirows)], lidx_v)
        _fm_chunks(fidx_v, sidx_v, fm_tab, fm_v, fm_out, sem_f, sem_s,
                   rpw // CH)

        def lin_chunk(c, carry):
            gcs = [pltpu.async_copy(lin_tab.at[lidx_v.at[c * LNJ + j]],
                                    lin_v.at[pl.ds(j * 128, 128)], sem_l)
                   for j in range(LNJ)]
            for cp in gcs:
                cp.wait()
            for g in range(SPC // 16):
                acc = jnp.zeros((16,), jnp.float32)
                for f in range(F):
                    acc = acc + lin_v[pl.ds(f * SPC + g * 16, 16)]
                ls_v[pl.ds(c * SPC + g * 16, 16)] = acc
            return carry

        lax.fori_loop(0, LNCH, lin_chunk, 0)
        pltpu.sync_copy(ls_v, lin_out.at[pl.ds(wid * SPW, SPW)])

    return _gather_a


@functools.cache
def _make_sc_gather_b():
    rpw = FB * B // NW
    irows = rpw // 128
    lirows = ROWS // NW // 128

    @functools.partial(
        pl.kernel,
        mesh=plsc.VectorSubcoreMesh(core_axis_name="c", subcore_axis_name="s"),
        out_type=jax.ShapeDtypeStruct((HROWS, K), jnp.float32),
        scratch_types=[
            pltpu.VMEM((irows, 128), jnp.int32),
            pltpu.VMEM((irows, 128), jnp.int32),
            pltpu.VMEM((CH, K), jnp.float32),
            pltpu.SemaphoreType.DMA,
            pltpu.SemaphoreType.DMA,
        ],
        compiler_params=pltpu.CompilerParams(use_tc_tiling_on_sc=False),
    )
    def _gather_b(fidx_hbm, sidx_hbm, fm_tab, fm_out,
                  fidx_v, sidx_v, fm_v, sem_f, sem_s):
        wid = lax.axis_index("s") * NC + lax.axis_index("c")
        pltpu.sync_copy(fidx_hbm.at[pl.ds(wid * irows, irows)], fidx_v)
        pltpu.sync_copy(sidx_hbm.at[pl.ds(wid * irows, irows)], sidx_v)
        _fm_chunks(fidx_v, sidx_v, fm_tab, fm_v, fm_out, sem_f, sem_s,
                   rpw // CH)

    return _gather_b


def _tc_body(fa_ref, fb_ref, xdt_ref, lin_ref, w0a_ref, w0b_ref, b0_ref,
             w1_ref, b1_ref, wo_ref, wd_ref, bd_ref, s_ref, out_ref):
    ya = fa_ref[...]
    yb = fb_ref[...]
    xdt = xdt_ref[...]
    w0a = w0a_ref[...]
    s_mat = s_ref[...]
    lane = lax.broadcasted_iota(jnp.int32, ya.shape[1:], 1)
    dn = (((0,), (0,)), ((), ()))
    h0 = lax.dot_general(xdt, w0b_ref[...], dimension_numbers=dn,
                         preferred_element_type=jnp.float32)
    sums = jnp.zeros((ya.shape[1], K), jnp.float32)
    sos = jnp.zeros((ya.shape[1], K), jnp.float32)
    planes = [ya[0], ya[1], yb[0], jnp.where(lane < 32, yb[1], 0.0)]
    for ct in range(4):
        x_ct = planes[ct]
        w_ct = w0a[128 * ct:128 * (ct + 1), :]
        s_ct = s_mat[128 * ct:128 * (ct + 1), :]
        h0 = h0 + jnp.dot(x_ct, w_ct, preferred_element_type=jnp.float32)
        sums = sums + jnp.dot(x_ct, s_ct, preferred_element_type=jnp.float32)
        sos = sos + jnp.dot(x_ct * x_ct, s_ct,
                            preferred_element_type=jnp.float32)
    h0 = jnp.maximum(h0 + b0_ref[...], 0.0)
    h1 = jnp.dot(h0, w1_ref[...], preferred_element_type=jnp.float32)
    h1 = jnp.maximum(h1 + b1_ref[...], 0.0)
    dnn = jnp.sum(h1 * wo_ref[...], axis=1, keepdims=True)
    cross = 0.5 * jnp.sum(sums * sums - sos, axis=1, keepdims=True)
    lind = lax.dot_general(xdt, wd_ref[...], dimension_numbers=dn,
                           preferred_element_type=jnp.float32)
    logit = lin_ref[...] + lind + cross + dnn + bd_ref[0, 0]
    out_ref[...] = jax.nn.sigmoid(logit)


def kernel(X_sparse, X_dense, lin_emb, fm_emb, W_dense, b_dense,
           W0, b0, W1, b1, W_out):
    xs = X_sparse.astype(jnp.int32)
    f_arange = np.arange(F, dtype=np.int32)
    qoff = jnp.asarray(((f_arange // 8) % 2) * (8 * V) + (f_arange % 8))
    flat_idx = (xs.T * 8 + qoff[:, None]).reshape(ROWS // 128, 128)
    fidx_a = flat_idx[:FA * B // 128]
    fidx_b = flat_idx[FA * B // 128:]
    loffs = jnp.arange(F, dtype=jnp.int32) * LTAB
    lin_idx = (xs.reshape(NW, LNCH, SPC, F).transpose(0, 1, 3, 2)
               + loffs[None, None, :, None]).reshape(ROWS // 128, 128)

    qf = fm_emb.transpose(0, 2, 1).reshape(F * K, V)
    xp = functools.partial(
        pl.pallas_call, _xpose_body,
        out_shape=jax.ShapeDtypeStruct((2, V, 128), jnp.float32))
    tab_a = xp(grid=(2, 13),
               in_specs=[pl.BlockSpec((128, 8192), lambda q, c: (q, c))],
               out_specs=pl.BlockSpec((1, 8192, 128), lambda q, c: (q, c, 0)),
               )(qf).reshape(2 * V * 8, K)
    tab_b = xp(grid=(2, 13),
               in_specs=[pl.BlockSpec((128, 8192), lambda q, c: (q + 2, c))],
               out_specs=pl.BlockSpec((1, 8192, 128), lambda q, c: (q, c, 0)),
               )(qf).reshape(2 * V * 8, K)
    l3 = lin_emb.transpose(0, 2, 1)
    lin_tab = pl.pallas_call(
        _lin_body,
        grid=(F,),
        in_specs=[pl.BlockSpec((1, 1, V), lambda f: (f, 0, 0))],
        out_specs=pl.BlockSpec((1, 784, 128), lambda f: (f, 0, 0)),
        out_shape=jax.ShapeDtypeStruct((F, 784, 128), jnp.float32),
    )(l3).reshape(F * LTAB)

    planes_a, lin_sum = _make_sc_gather_a()(
        fidx_a, jnp.asarray(_SIDX_A), lin_idx, tab_a, lin_tab)
    planes_b = _make_sc_gather_b()(fidx_b, jnp.asarray(_SIDX_B), tab_b)

    w0a = jnp.pad(W0[:, :F * K].T, ((0, 96), (0, 0)))
    w0b = W0[:, F * K:].T
    w1 = W1.T
    s_mat = jnp.asarray(np.vstack([
        np.tile(np.eye(K, dtype=np.float32), (F, 1)),
        np.zeros((96, K), np.float32)]))

    bB = 2048
    nb = B // bB
    out = pl.pallas_call(
        _tc_body,
        grid=(nb,),
        in_specs=[
            pl.BlockSpec((2, bB, 128), lambda i: (0, i, 0)),
            pl.BlockSpec((2, bB, 128), lambda i: (0, i, 0)),
            pl.BlockSpec((D, bB), lambda i: (0, i)),
            pl.BlockSpec((bB, 1), lambda i: (i, 0)),
            pl.BlockSpec((512, 256), lambda i: (0, 0)),
            pl.BlockSpec((D, 256), lambda i: (0, 0)),
            pl.BlockSpec((1, 256), lambda i: (0, 0)),
            pl.BlockSpec((256, 128), lambda i: (0, 0)),
            pl.BlockSpec((1, 128), lambda i: (0, 0)),
            pl.BlockSpec((1, 128), lambda i: (0, 0)),
            pl.BlockSpec((D, 1), lambda i: (0, 0)),
            pl.BlockSpec((1, 1), lambda i: (0, 0)),
            pl.BlockSpec((512, K), lambda i: (0, 0)),
        ],
        out_specs=pl.BlockSpec((bB, 1), lambda i: (i, 0)),
        out_shape=jax.ShapeDtypeStruct((B, 1), jnp.float32),
    )(planes_a.reshape(2, B, 128), planes_b.reshape(2, B, 128), X_dense.T,
      lin_sum.reshape(B, 1), w0a, w0b, b0.reshape(1, 256), w1,
      b1.reshape(1, 128), W_out, W_dense.T, b_dense.reshape(1, 1), s_mat)
    return out.reshape(B)
